# initial kernel scaffold (unmeasured)
import jax
import jax.numpy as jnp
from jax import lax
from jax.experimental import pallas as pl
from jax.experimental.pallas import tpu as pltpu

N_DEV = 4
SQ = 2048
H = 8
DH = 128
D = 1024
TAIL = 128
QB = 256
KW = 512
NQB = SQ // QB
SCALE = 0.08838834764831843
BF = jnp.bfloat16
F32 = jnp.float32
MESH = pl.DeviceIdType.MESH


def kernel(x, Wq, K_ext, V_ext, Wo):
    def body(x_hbm, wq_hbm, k_hbm, v_hbm, wo_hbm, out_ref,
             q_buf, wq_bf, wo_bf, k_main, v_main, k_tail, v_tail,
             ld_stage, kv_stage, send_buf, tail_stage, tail_send,
             ctx_buf, partials,
             copy_sem, send_sems, tail_send_sem, relay_send_sem,
             kmain_sem, vmain_sem, ktail_sem, vtail_sem,
             krelay_sem, vrelay_sem, ar_send_sems, ar_recv_sems):

        me = lax.axis_index("i")

        def local_copy(src, dst):
            cp = pltpu.make_async_copy(src, dst, copy_sem)
            cp.start()
            cp.wait()

        def stage_cast(hbm, d, dst_ref):
            for c in range(2):
                rows = pl.ds(1024 * c, 1024)
                local_copy(hbm.at[0, rows, pl.ds(8 * d, 8), :], kv_stage)
                dst_ref[rows, :, :] = kv_stage[...].astype(BF)

        def rdma(src, dst, ssem, rsem, dev):
            return pltpu.make_async_remote_copy(
                src_ref=src, dst_ref=dst, send_sem=ssem, recv_sem=rsem,
                device_id=(dev,), device_id_type=MESH)

        def k2_relay():
            return rdma(send_buf.at[0], send_buf.at[0], send_sems.at[0],
                        krelay_sem, 1)

        def v2_relay():
            return rdma(send_buf.at[1], send_buf.at[1], send_sems.at[1],
                        vrelay_sem, 3)

        def k1_send():
            return rdma(send_buf.at[0], k_main, send_sems.at[0], kmain_sem, 1)

        def v1_send():
            return rdma(send_buf.at[0], v_main, send_sems.at[0], vmain_sem, 1)

        def k3_send():
            return rdma(send_buf.at[1], k_main, send_sems.at[1], kmain_sem, 3)

        def v3_send():
            return rdma(send_buf.at[1], v_main, send_sems.at[1], vmain_sem, 3)

        @pl.when(me == 0)
        def _():
            stage_cast(k_hbm, 2, send_buf.at[0])
            k2_relay().start()
            stage_cast(v_hbm, 2, send_buf.at[1])
            v2_relay().start()

        @pl.when(me == 1)
        def _():
            local_copy(k_hbm.at[0, pl.ds(0, TAIL), :, :], tail_stage)
            kt = tail_stage[...].astype(BF)
            k_tail[...] = kt[:, 8:16, :]
            for d in (0, 2, 3):
                tail_send[...] = kt[:, 8 * d:8 * d + 8, :]
                snd = rdma(tail_send, k_tail, tail_send_sem, ktail_sem, d)
                snd.start()
                snd.wait_send()
            local_copy(v_hbm.at[0, pl.ds(0, TAIL), :, :], tail_stage)
            vt = tail_stage[...].astype(BF)
            v_tail[...] = vt[:, 8:16, :]
            for d in (0, 2, 3):
                tail_send[...] = vt[:, 8 * d:8 * d + 8, :]
                snd = rdma(tail_send, v_tail, tail_send_sem, vtail_sem, d)
                snd.start()
                snd.wait_send()

        for c in range(2):
            rows = pl.ds(512 * c, 512)
            local_copy(wq_hbm.at[rows, :], ld_stage)
            wq_bf[rows, :] = ld_stage[...].astype(BF)
        for c in range(4):
            rows = pl.ds(512 * c, 512)
            local_copy(x_hbm.at[0, rows, :], ld_stage)
            q_buf[rows, :] = lax.dot_general(
                ld_stage[...].astype(BF), wq_bf[...],
                (((1,), (0,)), ((), ())),
                preferred_element_type=F32).astype(BF)

        @pl.when(me == 0)
        def _():
            stage_cast(k_hbm, 0, k_main)
            stage_cast(v_hbm, 0, v_main)
            k2_relay().wait_send()
            stage_cast(k_hbm, 1, send_buf.at[0])
            k1_send().start()
            v2_relay().wait_send()
            stage_cast(k_hbm, 3, send_buf.at[1])
            k3_send().start()
            k1_send().wait_send()
            stage_cast(v_hbm, 1, send_buf.at[0])
            v1_send().start()
            k3_send().wait_send()
            stage_cast(v_hbm, 3, send_buf.at[1])
            v3_send().start()
            v1_send().wait_send()
            v3_send().wait_send()

        @pl.when(me == 1)
        def _():
            rdma(send_buf.at[0], send_buf.at[0], relay_send_sem,
                 krelay_sem, 0).wait_recv()
            fwd = rdma(send_buf.at[0], k_main, relay_send_sem, kmain_sem, 2)
            fwd.start()
            fwd.wait_send()

        @pl.when(me == 3)
        def _():
            rdma(send_buf.at[1], send_buf.at[1], relay_send_sem,
                 vrelay_sem, 0).wait_recv()
            fwd = rdma(send_buf.at[1], v_main, relay_send_sem, vmain_sem, 2)
            fwd.start()
            fwd.wait_send()

        @pl.when(me != 0)
        def _():
            rdma(send_buf.at[0], k_main, copy_sem, kmain_sem, 0).wait_recv()
            rdma(send_buf.at[0], v_main, copy_sem, vmain_sem, 0).wait_recv()

        @pl.when(me != 1)
        def _():
            rdma(tail_send, k_tail, copy_sem, ktail_sem, 1).wait_recv()
            rdma(tail_send, v_tail, copy_sem, vtail_sem, 1).wait_recv()

        for h in range(H):
            for qb in range(NQB):
                qs = QB * qb
                ks = max(0, qs - TAIL)
                if qb < NQB - 1:
                    k_win = k_main[pl.ds(ks, KW), h, :]
                    v_win = v_main[pl.ds(ks, KW), h, :]
                else:
                    k_win = jnp.concatenate(
                        [k_main[pl.ds(ks, KW - TAIL), h, :], k_tail[:, h, :]],
                        axis=0)
                    v_win = jnp.concatenate(
                        [v_main[pl.ds(ks, KW - TAIL), h, :], v_tail[:, h, :]],
                        axis=0)
                q_blk = q_buf[pl.ds(qs, QB), pl.ds(DH * h, DH)]
                s = lax.dot_general(q_blk, k_win, (((1,), (1,)), ((), ())),
                                    preferred_element_type=F32) * SCALE
                qi = qs + lax.broadcasted_iota(jnp.int32, (QB, KW), 0)
                ki = ks + lax.broadcasted_iota(jnp.int32, (QB, KW), 1)
                s = jnp.where(jnp.abs(qi - ki) <= TAIL, s, -1e9)
                m = jnp.max(s, axis=1, keepdims=True)
                e = jnp.exp(s - m)
                p = (e / jnp.sum(e, axis=1, keepdims=True)).astype(BF)
                ctx_blk = lax.dot_general(p, v_win, (((1,), (0,)), ((), ())),
                                          preferred_element_type=F32)
                ctx_buf[pl.ds(qs, QB), pl.ds(DH * h, DH)] = ctx_blk.astype(BF)

        for c in range(2):
            rows = pl.ds(512 * c, 512)
            local_copy(wo_hbm.at[rows, :], ld_stage)
            wo_bf[rows, :] = ld_stage[...].astype(BF)
        for r in range(4):
            rows = pl.ds(512 * r, 512)
            pr = lax.dot_general(ctx_buf[rows, :], wo_bf[...],
                                 (((1,), (0,)), ((), ())),
                                 preferred_element_type=F32)
            out_ref[0, rows, :] = pr
            partials[0, rows, :] = pr.astype(BF)

        right = lax.rem(me + 1, N_DEV)
        for hop in range(N_DEV - 1):
            s_slot, r_slot = hop % 2, (hop + 1) % 2
            step = pltpu.make_async_remote_copy(
                src_ref=partials.at[s_slot], dst_ref=partials.at[r_slot],
                send_sem=ar_send_sems.at[hop], recv_sem=ar_recv_sems.at[hop],
                device_id=(right,), device_id_type=MESH)
            step.start()
            step.wait()
            for r in range(4):
                rows = pl.ds(512 * r, 512)
                out_ref[0, rows, :] = (out_ref[0, rows, :]
                                       + partials[r_slot, rows, :].astype(F32))

    return pl.pallas_call(
        body,
        out_shape=jax.ShapeDtypeStruct((1, SQ, D), F32),
        in_specs=[pl.BlockSpec(memory_space=pltpu.ANY)] * 5,
        out_specs=pl.BlockSpec(memory_space=pltpu.VMEM),
        scratch_shapes=[
            pltpu.VMEM((SQ, D), BF),
            pltpu.VMEM((D, D), BF),
            pltpu.VMEM((D, D), BF),
            pltpu.VMEM((SQ, H, DH), BF),
            pltpu.VMEM((SQ, H, DH), BF),
            pltpu.VMEM((TAIL, H, DH), BF),
            pltpu.VMEM((TAIL, H, DH), BF),
            pltpu.VMEM((512, D), F32),
            pltpu.VMEM((1024, H, DH), F32),
            pltpu.VMEM((2, SQ, H, DH), BF),
            pltpu.VMEM((TAIL, 32, DH), F32),
            pltpu.VMEM((TAIL, H, DH), BF),
            pltpu.VMEM((SQ, D), BF),
            pltpu.VMEM((2, SQ, D), BF),
            pltpu.SemaphoreType.DMA,
            pltpu.SemaphoreType.DMA((2,)),
            pltpu.SemaphoreType.DMA,
            pltpu.SemaphoreType.DMA,
            pltpu.SemaphoreType.DMA,
            pltpu.SemaphoreType.DMA,
            pltpu.SemaphoreType.DMA,
            pltpu.SemaphoreType.DMA,
            pltpu.SemaphoreType.DMA,
            pltpu.SemaphoreType.DMA,
            pltpu.SemaphoreType.DMA((3,)),
            pltpu.SemaphoreType.DMA((3,)),
        ],
        compiler_params=pltpu.CompilerParams(collective_id=0),
    )(x, Wq, K_ext, V_ext, Wo)


# baseline (device time: 373948 ns/iter reference)
import jax
import jax.numpy as jnp
from jax import lax
from jax.experimental import pallas as pl
from jax.experimental.pallas import tpu as pltpu

N_DEV = 4
SQ = 2048
H = 8
DH = 128
D = 1024
TAIL = 128
QB = 256
KW = 512
NQB = SQ // QB
SCALE = 0.08838834764831843
BF = jnp.bfloat16
F32 = jnp.float32
MESH = pl.DeviceIdType.MESH


def kernel(x, Wq, K_ext, V_ext, Wo):
    def body(x_hbm, wq_hbm, k_hbm, v_hbm, wo_hbm, out_ref,
             q_buf, wq_bf, wo_bf, k_main, v_main, k_tail, v_tail,
             ld_stage, kv_stage, send_buf, tail_stage, tail_send,
             ctx_buf, partials,
             copy_sem, send_sems, tail_send_sem, relay_send_sem,
             kmain_sem, vmain_sem, ktail_sem, vtail_sem,
             krelay_sem, vrelay_sem, ar_send_sems, ar_recv_sems):

        me = lax.axis_index("i")

        def local_copy(src, dst):
            cp = pltpu.make_async_copy(src, dst, copy_sem)
            cp.start()
            cp.wait()

        def stage_cast(hbm, d, dst_ref):
            for c in range(2):
                rows = pl.ds(1024 * c, 1024)
                local_copy(hbm.at[0, rows, pl.ds(8 * d, 8), :], kv_stage)
                dst_ref[rows, :, :] = kv_stage[...].astype(BF)

        def rdma(src, dst, ssem, rsem, dev):
            return pltpu.make_async_remote_copy(
                src_ref=src, dst_ref=dst, send_sem=ssem, recv_sem=rsem,
                device_id=(dev,), device_id_type=MESH)

        def k2_relay():
            return rdma(send_buf.at[0], send_buf.at[0], send_sems.at[0],
                        krelay_sem, 1)

        def v2_relay():
            return rdma(send_buf.at[1], send_buf.at[1], send_sems.at[1],
                        vrelay_sem, 3)

        def k1_send():
            return rdma(send_buf.at[0], k_main, send_sems.at[0], kmain_sem, 1)

        def v1_send():
            return rdma(send_buf.at[0], v_main, send_sems.at[0], vmain_sem, 1)

        def k3_send():
            return rdma(send_buf.at[1], k_main, send_sems.at[1], kmain_sem, 3)

        def v3_send():
            return rdma(send_buf.at[1], v_main, send_sems.at[1], vmain_sem, 3)

        @pl.when(me == 0)
        def _():
            stage_cast(k_hbm, 2, send_buf.at[0])
            k2_relay().start()
            stage_cast(v_hbm, 2, send_buf.at[1])
            v2_relay().start()

        @pl.when(me == 1)
        def _():
            local_copy(k_hbm.at[0, pl.ds(0, TAIL), :, :], tail_stage)
            kt = tail_stage[...].astype(BF)
            k_tail[...] = kt[:, 8:16, :]
            for d in (0, 2, 3):
                tail_send[...] = kt[:, 8 * d:8 * d + 8, :]
                snd = rdma(tail_send, k_tail, tail_send_sem, ktail_sem, d)
                snd.start()
                snd.wait_send()
            local_copy(v_hbm.at[0, pl.ds(0, TAIL), :, :], tail_stage)
            vt = tail_stage[...].astype(BF)
            v_tail[...] = vt[:, 8:16, :]
            for d in (0, 2, 3):
                tail_send[...] = vt[:, 8 * d:8 * d + 8, :]
                snd = rdma(tail_send, v_tail, tail_send_sem, vtail_sem, d)
                snd.start()
                snd.wait_send()

        for c in range(2):
            rows = pl.ds(512 * c, 512)
            local_copy(wq_hbm.at[rows, :], ld_stage)
            wq_bf[rows, :] = ld_stage[...].astype(BF)
        for c in range(4):
            rows = pl.ds(512 * c, 512)
            local_copy(x_hbm.at[0, rows, :], ld_stage)
            q_buf[rows, :] = lax.dot_general(
                ld_stage[...].astype(BF), wq_bf[...],
                (((1,), (0,)), ((), ())),
                preferred_element_type=F32).astype(BF)

        @pl.when(me == 0)
        def _():
            stage_cast(k_hbm, 0, k_main)
            stage_cast(v_hbm, 0, v_main)
            k2_relay().wait_send()
            stage_cast(k_hbm, 1, send_buf.at[0])
            k1_send().start()
            v2_relay().wait_send()
            stage_cast(k_hbm, 3, send_buf.at[1])
            k3_send().start()
            k1_send().wait_send()
            stage_cast(v_hbm, 1, send_buf.at[0])
            v1_send().start()
            k3_send().wait_send()
            stage_cast(v_hbm, 3, send_buf.at[1])
            v3_send().start()
            v1_send().wait_send()
            v3_send().wait_send()

        @pl.when(me == 1)
        def _():
            rdma(send_buf.at[0], send_buf.at[0], relay_send_sem,
                 krelay_sem, 0).wait_recv()
            fwd = rdma(send_buf.at[0], k_main, relay_send_sem, kmain_sem, 2)
            fwd.start()
            fwd.wait_send()

        @pl.when(me == 3)
        def _():
            rdma(send_buf.at[1], send_buf.at[1], relay_send_sem,
                 vrelay_sem, 0).wait_recv()
            fwd = rdma(send_buf.at[1], v_main, relay_send_sem, vmain_sem, 2)
            fwd.start()
            fwd.wait_send()

        @pl.when(me != 0)
        def _():
            rdma(send_buf.at[0], k_main, copy_sem, kmain_sem, 0).wait_recv()
            rdma(send_buf.at[0], v_main, copy_sem, vmain_sem, 0).wait_recv()

        @pl.when(me != 1)
        def _():
            rdma(tail_send, k_tail, copy_sem, ktail_sem, 1).wait_recv()
            rdma(tail_send, v_tail, copy_sem, vtail_sem, 1).wait_recv()

        for h in range(H):
            for qb in range(NQB):
                qs = QB * qb
                ks = max(0, qs - TAIL)
                if qb < NQB - 1:
                    k_win = k_main[pl.ds(ks, KW), h, :]
                    v_win = v_main[pl.ds(ks, KW), h, :]
                else:
                    k_win = jnp.concatenate(
                        [k_main[pl.ds(ks, KW - TAIL), h, :], k_tail[:, h, :]],
                        axis=0)
                    v_win = jnp.concatenate(
                        [v_main[pl.ds(ks, KW - TAIL), h, :], v_tail[:, h, :]],
                        axis=0)
                q_blk = q_buf[pl.ds(qs, QB), pl.ds(DH * h, DH)]
                s = lax.dot_general(q_blk, k_win, (((1,), (1,)), ((), ())),
                                    preferred_element_type=F32) * SCALE
                qi = qs + lax.broadcasted_iota(jnp.int32, (QB, KW), 0)
                ki = ks + lax.broadcasted_iota(jnp.int32, (QB, KW), 1)
                s = jnp.where(jnp.abs(qi - ki) <= TAIL, s, -1e9)
                m = jnp.max(s, axis=1, keepdims=True)
                e = jnp.exp(s - m)
                p = (e / jnp.sum(e, axis=1, keepdims=True)).astype(BF)
                ctx_blk = lax.dot_general(p, v_win, (((1,), (0,)), ((), ())),
                                          preferred_element_type=F32)
                ctx_buf[pl.ds(qs, QB), pl.ds(DH * h, DH)] = ctx_blk.astype(BF)

        for c in range(2):
            rows = pl.ds(512 * c, 512)
            local_copy(wo_hbm.at[rows, :], ld_stage)
            wo_bf[rows, :] = ld_stage[...].astype(BF)
        for r in range(4):
            rows = pl.ds(512 * r, 512)
            pr = lax.dot_general(ctx_buf[rows, :], wo_bf[...],
                                 (((1,), (0,)), ((), ())),
                                 preferred_element_type=F32)
            out_ref[0, rows, :] = pr
            partials[0, rows, :] = pr.astype(BF)

        right = lax.rem(me + 1, N_DEV)
        for hop in range(N_DEV - 1):
            s_slot, r_slot = hop % 2, (hop + 1) % 2
            step = pltpu.make_async_remote_copy(
                src_ref=partials.at[s_slot], dst_ref=partials.at[r_slot],
                send_sem=ar_send_sems.at[hop], recv_sem=ar_recv_sems.at[hop],
                device_id=(right,), device_id_type=MESH)
            step.start()
            step.wait()
            for r in range(4):
                rows = pl.ds(512 * r, 512)
                out_ref[0, rows, :] = (out_ref[0, rows, :]
                                       + partials[r_slot, rows, :].astype(F32))

    return pl.pallas_call(
        body,
        out_shape=jax.ShapeDtypeStruct((1, SQ, D), F32),
        in_specs=[pl.BlockSpec(memory_space=pl.ANY)] * 5,
        out_specs=pl.BlockSpec(memory_space=pltpu.MemorySpace.VMEM),
        scratch_shapes=[
            pltpu.VMEM((SQ, D), BF),
            pltpu.VMEM((D, D), BF),
            pltpu.VMEM((D, D), BF),
            pltpu.VMEM((SQ, H, DH), BF),
            pltpu.VMEM((SQ, H, DH), BF),
            pltpu.VMEM((TAIL, H, DH), BF),
            pltpu.VMEM((TAIL, H, DH), BF),
            pltpu.VMEM((512, D), F32),
            pltpu.VMEM((1024, H, DH), F32),
            pltpu.VMEM((2, SQ, H, DH), BF),
            pltpu.VMEM((TAIL, 32, DH), F32),
            pltpu.VMEM((TAIL, H, DH), BF),
            pltpu.VMEM((SQ, D), BF),
            pltpu.VMEM((2, SQ, D), BF),
            pltpu.SemaphoreType.DMA,
            pltpu.SemaphoreType.DMA((2,)),
            pltpu.SemaphoreType.DMA,
            pltpu.SemaphoreType.DMA,
            pltpu.SemaphoreType.DMA,
            pltpu.SemaphoreType.DMA,
            pltpu.SemaphoreType.DMA,
            pltpu.SemaphoreType.DMA,
            pltpu.SemaphoreType.DMA,
            pltpu.SemaphoreType.DMA,
            pltpu.SemaphoreType.DMA((3,)),
            pltpu.SemaphoreType.DMA((3,)),
        ],
        compiler_params=pltpu.CompilerParams(
            vmem_limit_bytes=63 * 1024 * 1024),
    )(x, Wq, K_ext, V_ext, Wo)


# device time: 275722 ns/iter; 1.3563x vs baseline; 1.3563x over previous
import jax
import jax.numpy as jnp
from jax import lax
from jax.experimental import pallas as pl
from jax.experimental.pallas import tpu as pltpu

N_DEV = 4
SQ = 2048
H = 8
DH = 128
D = 1024
TAIL = 128
QB = 256
KW = 512
NQB = SQ // QB
SCALE = 0.08838834764831843
BF = jnp.bfloat16
F32 = jnp.float32
MESH = pl.DeviceIdType.MESH


def kernel(x, Wq, K_ext, V_ext, Wo):
    def body(x_hbm, wq_hbm, k_hbm, v_hbm, wo_hbm, out_ref,
             q_buf, wq_bf, wo_bf, k_main, v_main, k_tail, v_tail,
             ld_stage, kv_stage, send_buf, tail_stage, tail_send,
             ctx_buf, p_r, p_l, recv_r, recv_l, acc_r, acc_l,
             copy_sem, send_sems, tail_send_sem, relay_send_sem,
             kmain_sem, vmain_sem, ktail_sem, vtail_sem,
             krelay_sem, vrelay_sem,
             ars_send_r, ars_recv_r, ars_send_l, ars_recv_l):

        me = lax.axis_index("i")

        def local_copy(src, dst):
            cp = pltpu.make_async_copy(src, dst, copy_sem)
            cp.start()
            cp.wait()

        def stage_cast(hbm, d, dst_ref):
            for c in range(2):
                rows = pl.ds(1024 * c, 1024)
                local_copy(hbm.at[0, rows, pl.ds(8 * d, 8), :], kv_stage)
                dst_ref[rows, :, :] = kv_stage[...].astype(BF)

        def rdma(src, dst, ssem, rsem, dev):
            return pltpu.make_async_remote_copy(
                src_ref=src, dst_ref=dst, send_sem=ssem, recv_sem=rsem,
                device_id=(dev,), device_id_type=MESH)

        def k2_relay():
            return rdma(send_buf.at[0], send_buf.at[0], send_sems.at[0],
                        krelay_sem, 1)

        def v2_relay():
            return rdma(send_buf.at[1], send_buf.at[1], send_sems.at[1],
                        vrelay_sem, 3)

        def k1_send():
            return rdma(send_buf.at[0], k_main, send_sems.at[0], kmain_sem, 1)

        def v1_send():
            return rdma(send_buf.at[0], v_main, send_sems.at[0], vmain_sem, 1)

        def k3_send():
            return rdma(send_buf.at[1], k_main, send_sems.at[1], kmain_sem, 3)

        def v3_send():
            return rdma(send_buf.at[1], v_main, send_sems.at[1], vmain_sem, 3)

        @pl.when(me == 0)
        def _():
            stage_cast(k_hbm, 2, send_buf.at[0])
            k2_relay().start()
            stage_cast(v_hbm, 2, send_buf.at[1])
            v2_relay().start()

        @pl.when(me == 1)
        def _():
            local_copy(k_hbm.at[0, pl.ds(0, TAIL), :, :], tail_stage)
            kt = tail_stage[...].astype(BF)
            k_tail[...] = kt[:, 8:16, :]
            for d in (0, 2, 3):
                tail_send[...] = kt[:, 8 * d:8 * d + 8, :]
                snd = rdma(tail_send, k_tail, tail_send_sem, ktail_sem, d)
                snd.start()
                snd.wait_send()
            local_copy(v_hbm.at[0, pl.ds(0, TAIL), :, :], tail_stage)
            vt = tail_stage[...].astype(BF)
            v_tail[...] = vt[:, 8:16, :]
            for d in (0, 2, 3):
                tail_send[...] = vt[:, 8 * d:8 * d + 8, :]
                snd = rdma(tail_send, v_tail, tail_send_sem, vtail_sem, d)
                snd.start()
                snd.wait_send()

        for c in range(2):
            rows = pl.ds(512 * c, 512)
            local_copy(wq_hbm.at[rows, :], ld_stage)
            wq_bf[rows, :] = ld_stage[...].astype(BF)
        for c in range(4):
            rows = pl.ds(512 * c, 512)
            local_copy(x_hbm.at[0, rows, :], ld_stage)
            q_buf[rows, :] = lax.dot_general(
                ld_stage[...].astype(BF), wq_bf[...],
                (((1,), (0,)), ((), ())),
                preferred_element_type=F32).astype(BF)
        for c in range(2):
            rows = pl.ds(512 * c, 512)
            local_copy(wo_hbm.at[rows, :], ld_stage)
            wo_bf[rows, :] = ld_stage[...].astype(BF)

        @pl.when(me == 0)
        def _():
            stage_cast(k_hbm, 0, k_main)
            stage_cast(v_hbm, 0, v_main)
            k2_relay().wait_send()
            stage_cast(k_hbm, 1, send_buf.at[0])
            k1_send().start()
            v2_relay().wait_send()
            stage_cast(k_hbm, 3, send_buf.at[1])
            k3_send().start()
            k1_send().wait_send()
            stage_cast(v_hbm, 1, send_buf.at[0])
            v1_send().start()
            k3_send().wait_send()
            stage_cast(v_hbm, 3, send_buf.at[1])
            v3_send().start()
            v1_send().wait_send()
            v3_send().wait_send()

        @pl.when(me == 1)
        def _():
            rdma(send_buf.at[0], send_buf.at[0], relay_send_sem,
                 krelay_sem, 0).wait_recv()
            fwd = rdma(send_buf.at[0], k_main, relay_send_sem, kmain_sem, 2)
            fwd.start()
            fwd.wait_send()

        @pl.when(me == 3)
        def _():
            rdma(send_buf.at[1], send_buf.at[1], relay_send_sem,
                 vrelay_sem, 0).wait_recv()
            fwd = rdma(send_buf.at[1], v_main, relay_send_sem, vmain_sem, 2)
            fwd.start()
            fwd.wait_send()

        @pl.when(me != 0)
        def _():
            rdma(send_buf.at[0], k_main, copy_sem, kmain_sem, 0).wait_recv()
            rdma(send_buf.at[0], v_main, copy_sem, vmain_sem, 0).wait_recv()

        @pl.when(me != 1)
        def _():
            rdma(tail_send, k_tail, copy_sem, ktail_sem, 1).wait_recv()
            rdma(tail_send, v_tail, copy_sem, vtail_sem, 1).wait_recv()

        for h in range(H):
            for qb in range(NQB):
                qs = QB * qb
                ks = max(0, qs - TAIL)
                if qb < NQB - 1:
                    k_win = k_main[pl.ds(ks, KW), h, :]
                    v_win = v_main[pl.ds(ks, KW), h, :]
                else:
                    k_win = jnp.concatenate(
                        [k_main[pl.ds(ks, KW - TAIL), h, :], k_tail[:, h, :]],
                        axis=0)
                    v_win = jnp.concatenate(
                        [v_main[pl.ds(ks, KW - TAIL), h, :], v_tail[:, h, :]],
                        axis=0)
                q_blk = q_buf[pl.ds(qs, QB), pl.ds(DH * h, DH)]
                s = lax.dot_general(q_blk, k_win, (((1,), (1,)), ((), ())),
                                    preferred_element_type=F32) * SCALE
                qi = qs + lax.broadcasted_iota(jnp.int32, (QB, KW), 0)
                ki = ks + lax.broadcasted_iota(jnp.int32, (QB, KW), 1)
                s = jnp.where(jnp.abs(qi - ki) <= TAIL, s, -1e9)
                m = jnp.max(s, axis=1, keepdims=True)
                e = jnp.exp(s - m)
                p = (e / jnp.sum(e, axis=1, keepdims=True)).astype(BF)
                ctx_blk = lax.dot_general(p, v_win, (((1,), (0,)), ((), ())),
                                          preferred_element_type=F32)
                ctx_buf[pl.ds(qs, QB), pl.ds(DH * h, DH)] = ctx_blk.astype(BF)

        for r in range(4):
            rows = pl.ds(512 * r, 512)
            pr = lax.dot_general(ctx_buf[rows, :], wo_bf[...],
                                 (((1,), (0,)), ((), ())),
                                 preferred_element_type=F32)
            p_r[rows, :] = pr[:, 0:512].astype(BF)
            p_l[rows, :] = pr[:, 512:].astype(BF)

        right = lax.rem(me + 1, N_DEV)
        left = lax.rem(me + 3, N_DEV)

        def crows(c):
            return pl.ds(c * 512, 512)

        rings = ((p_r, recv_r, acc_r, ars_send_r, ars_recv_r, right, False),
                 (p_l, recv_l, acc_l, ars_send_l, ars_recv_l, left, True))
        inflight = []

        def ar_step(p, rv, ac, ssem, rsem, dev, step_idx, src, c_dst):
            snd = pltpu.make_async_remote_copy(
                src_ref=src, dst_ref=rv.at[crows(c_dst)],
                send_sem=ssem.at[step_idx], recv_sem=rsem.at[step_idx],
                device_id=(dev,), device_id_type=MESH)
            snd.start()
            inflight.append(snd)
            return snd

        def ar_wait_recv(rv, rsem, step_idx, c_in):
            pltpu.make_async_remote_copy(
                src_ref=rv.at[crows(c_in)], dst_ref=rv.at[crows(c_in)],
                send_sem=ars_send_r.at[step_idx], recv_sem=rsem.at[step_idx],
                device_id=(0,), device_id_type=MESH).wait_recv()

        for s in range(3):
            for (p, rv, ac, ssem, rsem, dev, is_l) in rings:
                c = lax.rem(me + s, 4) if is_l else lax.rem(me + 4 - s, 4)
                if s == 0:
                    ar_step(p, rv, ac, ssem, rsem, dev, 0, p.at[crows(c)], c)
                else:
                    c_prev = c
                    ar_wait_recv(rv, rsem, s - 1, c_prev)
                    ac[s - 1, :, :] = p[crows(c), :] + rv[crows(c), :]
                    ar_step(p, rv, ac, ssem, rsem, dev, s, ac.at[s - 1], c)
        for (p, rv, ac, ssem, rsem, dev, is_l) in rings:
            c_fin = lax.rem(me + 3, 4) if is_l else lax.rem(me + 1, 4)
            ar_wait_recv(rv, rsem, 2, c_fin)
            rv[crows(c_fin), :] = p[crows(c_fin), :] + rv[crows(c_fin), :]
        for s in range(3):
            for (p, rv, ac, ssem, rsem, dev, is_l) in rings:
                c = (lax.rem(me + 3 + s, 4) if is_l
                     else lax.rem(me + 5 - s, 4))
                if s > 0:
                    ar_wait_recv(rv, rsem, 3 + s - 1, c)
                ar_step(p, rv, ac, ssem, rsem, dev, 3 + s, rv.at[crows(c)], c)
        for (p, rv, ac, ssem, rsem, dev, is_l) in rings:
            c_last = lax.rem(me + 6, 4) if is_l else lax.rem(me + 2, 4)
            ar_wait_recv(rv, rsem, 5, c_last)
        for snd in inflight:
            snd.wait_send()

        for r in range(4):
            rows = pl.ds(512 * r, 512)
            out_ref[0, rows, 0:512] = recv_r[rows, :].astype(F32)
            out_ref[0, rows, 512:1024] = recv_l[rows, :].astype(F32)

    return pl.pallas_call(
        body,
        out_shape=jax.ShapeDtypeStruct((1, SQ, D), F32),
        in_specs=[pl.BlockSpec(memory_space=pl.ANY)] * 5,
        out_specs=pl.BlockSpec(memory_space=pltpu.MemorySpace.VMEM),
        scratch_shapes=[
            pltpu.VMEM((SQ, D), BF),
            pltpu.VMEM((D, D), BF),
            pltpu.VMEM((D, D), BF),
            pltpu.VMEM((SQ, H, DH), BF),
            pltpu.VMEM((SQ, H, DH), BF),
            pltpu.VMEM((TAIL, H, DH), BF),
            pltpu.VMEM((TAIL, H, DH), BF),
            pltpu.VMEM((512, D), F32),
            pltpu.VMEM((1024, H, DH), F32),
            pltpu.VMEM((2, SQ, H, DH), BF),
            pltpu.VMEM((TAIL, 32, DH), F32),
            pltpu.VMEM((TAIL, H, DH), BF),
            pltpu.VMEM((SQ, D), BF),
            pltpu.VMEM((SQ, 512), BF),
            pltpu.VMEM((SQ, 512), BF),
            pltpu.VMEM((SQ, 512), BF),
            pltpu.VMEM((SQ, 512), BF),
            pltpu.VMEM((2, 512, 512), BF),
            pltpu.VMEM((2, 512, 512), BF),
            pltpu.SemaphoreType.DMA,
            pltpu.SemaphoreType.DMA((2,)),
            pltpu.SemaphoreType.DMA,
            pltpu.SemaphoreType.DMA,
            pltpu.SemaphoreType.DMA,
            pltpu.SemaphoreType.DMA,
            pltpu.SemaphoreType.DMA,
            pltpu.SemaphoreType.DMA,
            pltpu.SemaphoreType.DMA,
            pltpu.SemaphoreType.DMA,
            pltpu.SemaphoreType.DMA((6,)),
            pltpu.SemaphoreType.DMA((6,)),
            pltpu.SemaphoreType.DMA((6,)),
            pltpu.SemaphoreType.DMA((6,)),
        ],
        compiler_params=pltpu.CompilerParams(
            vmem_limit_bytes=63 * 1024 * 1024),
    )(x, Wq, K_ext, V_ext, Wo)


# device time: 229838 ns/iter; 1.6270x vs baseline; 1.1996x over previous
import jax
import jax.numpy as jnp
from jax import lax
from jax.experimental import pallas as pl
from jax.experimental.pallas import tpu as pltpu

N_DEV = 4
SQ = 2048
H = 8
DH = 128
D = 1024
TAIL = 128
QB = 256
KW = 512
NQB = SQ // QB
SUB = 512
SCALE = 0.08838834764831843
BF = jnp.bfloat16
F32 = jnp.float32
MESH = pl.DeviceIdType.MESH

QBS_OF_SUB = {0: (0,), 1: (1, 2), 2: (3, 4), 3: (5, 6, 7)}


def kernel(x, Wq, K_ext, V_ext, Wo):
    def body(x_hbm, wq_hbm, k_hbm, v_hbm, wo_hbm, out_ref,
             q_buf, wq_bf, wo_bf, k_main, v_main, k_tail, v_tail,
             ld_stage, kv_stage, send_buf, tail_stage, tail_send,
             ctx_buf, p_r, p_l, recv_r, recv_l, acc_r, acc_l,
             copy_sem, send_sems, tail_send_sem, relay_send_sems,
             kmain_sems, vmain_sems, ktail_sem, vtail_sem,
             krelay_sems, vrelay_sems,
             ars_send_r, ars_recv_r, ars_send_l, ars_recv_l):

        me = lax.axis_index("i")

        def local_copy(src, dst):
            cp = pltpu.make_async_copy(src, dst, copy_sem)
            cp.start()
            cp.wait()

        def rdma(src, dst, ssem, rsem, dev):
            return pltpu.make_async_remote_copy(
                src_ref=src, dst_ref=dst, send_sem=ssem, recv_sem=rsem,
                device_id=(dev,), device_id_type=MESH)

        def stage_sub(hbm, d, slot, r):
            rows = pl.ds(SUB * r, SUB)
            local_copy(hbm.at[0, rows, pl.ds(8 * d, 8), :], kv_stage)
            send_buf[slot, rows, :, :] = kv_stage[...].astype(BF)

        def fill_sub(hbm, dst, r):
            rows = pl.ds(SUB * r, SUB)
            local_copy(hbm.at[0, rows, pl.ds(0, 8), :], kv_stage)
            dst[rows, :, :] = kv_stage[...].astype(BF)

        inflight = set()

        def wait_slot(slot, r):
            rows = pl.ds(SUB * r, SUB)
            rdma(send_buf.at[slot, rows, :, :], k_main.at[rows, :, :],
                 send_sems.at[slot, r], kmain_sems.at[r], 1).wait_send()

        def piece_sub(hbm, d, slot, r, dst_fn, rsem_arr, dev):
            if (slot, r) in inflight:
                wait_slot(slot, r)
            stage_sub(hbm, d, slot, r)
            rows = pl.ds(SUB * r, SUB)
            snd = rdma(send_buf.at[slot, rows, :, :], dst_fn(rows),
                       send_sems.at[slot, r], rsem_arr.at[r], dev)
            snd.start()
            inflight.add((slot, r))

        @pl.when(me == 0)
        def _():
            for r in range(4):
                piece_sub(k_hbm, 2, 0, r,
                          lambda rows: send_buf.at[0, rows, :, :],
                          krelay_sems, 1)
                piece_sub(v_hbm, 2, 1, r,
                          lambda rows: send_buf.at[1, rows, :, :],
                          vrelay_sems, 3)

        @pl.when(me == 1)
        def _():
            local_copy(k_hbm.at[0, pl.ds(0, TAIL), :, :], tail_stage)
            kt = tail_stage[...].astype(BF)
            k_tail[...] = kt[:, 8:16, :]
            for d in (0, 2, 3):
                tail_send[...] = kt[:, 8 * d:8 * d + 8, :]
                snd = rdma(tail_send, k_tail, tail_send_sem, ktail_sem, d)
                snd.start()
                snd.wait_send()
            local_copy(v_hbm.at[0, pl.ds(0, TAIL), :, :], tail_stage)
            vt = tail_stage[...].astype(BF)
            v_tail[...] = vt[:, 8:16, :]
            for d in (0, 2, 3):
                tail_send[...] = vt[:, 8 * d:8 * d + 8, :]
                snd = rdma(tail_send, v_tail, tail_send_sem, vtail_sem, d)
                snd.start()
                snd.wait_send()

        @pl.when(me == 1)
        def _():
            for r in range(4):
                rows = pl.ds(SUB * r, SUB)
                rdma(send_buf.at[0, rows, :, :], send_buf.at[0, rows, :, :],
                     relay_send_sems.at[r], krelay_sems.at[r], 0).wait_recv()
                rdma(send_buf.at[0, rows, :, :], k_main.at[rows, :, :],
                     relay_send_sems.at[r], kmain_sems.at[r], 2).start()
            for r in range(4):
                rows = pl.ds(SUB * r, SUB)
                rdma(send_buf.at[0, rows, :, :], k_main.at[rows, :, :],
                     relay_send_sems.at[r], kmain_sems.at[r], 2).wait_send()

        @pl.when(me == 3)
        def _():
            for r in range(4):
                rows = pl.ds(SUB * r, SUB)
                rdma(send_buf.at[1, rows, :, :], send_buf.at[1, rows, :, :],
                     relay_send_sems.at[r], vrelay_sems.at[r], 0).wait_recv()
                rdma(send_buf.at[1, rows, :, :], v_main.at[rows, :, :],
                     relay_send_sems.at[r], vmain_sems.at[r], 2).start()
            for r in range(4):
                rows = pl.ds(SUB * r, SUB)
                rdma(send_buf.at[1, rows, :, :], v_main.at[rows, :, :],
                     relay_send_sems.at[r], vmain_sems.at[r], 2).wait_send()

        for c in range(2):
            rows = pl.ds(512 * c, 512)
            local_copy(wq_hbm.at[rows, :], ld_stage)
            wq_bf[rows, :] = ld_stage[...].astype(BF)
        for c in range(4):
            rows = pl.ds(512 * c, 512)
            local_copy(x_hbm.at[0, rows, :], ld_stage)
            q_buf[rows, :] = lax.dot_general(
                ld_stage[...].astype(BF), wq_bf[...],
                (((1,), (0,)), ((), ())),
                preferred_element_type=F32).astype(BF)
        for c in range(2):
            rows = pl.ds(512 * c, 512)
            local_copy(wo_hbm.at[rows, :], ld_stage)
            wo_bf[rows, :] = ld_stage[...].astype(BF)

        @pl.when(me == 0)
        def _():
            k_dst = lambda rows: k_main.at[rows, :, :]
            v_dst = lambda rows: v_main.at[rows, :, :]
            for r in range(4):
                piece_sub(k_hbm, 1, 0, r, k_dst, kmain_sems, 1)
                piece_sub(k_hbm, 3, 1, r, k_dst, kmain_sems, 3)
                fill_sub(k_hbm, k_main, r)
            for r in range(4):
                piece_sub(v_hbm, 1, 0, r, v_dst, vmain_sems, 1)
                piece_sub(v_hbm, 3, 1, r, v_dst, vmain_sems, 3)
                fill_sub(v_hbm, v_main, r)

        @pl.when(me != 0)
        def _():
            for r in range(4):
                rows = pl.ds(SUB * r, SUB)
                rdma(send_buf.at[0, rows, :, :], k_main.at[rows, :, :],
                     copy_sem, kmain_sems.at[r], 0).wait_recv()

        @pl.when(me != 1)
        def _():
            rdma(tail_send, k_tail, copy_sem, ktail_sem, 1).wait_recv()
            rdma(tail_send, v_tail, copy_sem, vtail_sem, 1).wait_recv()

        for r in range(4):
            @pl.when(me != 0)
            def _(r=r):
                rows = pl.ds(SUB * r, SUB)
                rdma(send_buf.at[0, rows, :, :], v_main.at[rows, :, :],
                     copy_sem, vmain_sems.at[r], 0).wait_recv()

            for qb in QBS_OF_SUB[r]:
                qs = QB * qb
                ks = max(0, qs - TAIL)
                for h in range(H):
                    if qb < NQB - 1:
                        k_win = k_main[pl.ds(ks, KW), h, :]
                        v_win = v_main[pl.ds(ks, KW), h, :]
                    else:
                        k_win = jnp.concatenate(
                            [k_main[pl.ds(ks, KW - TAIL), h, :],
                             k_tail[:, h, :]], axis=0)
                        v_win = jnp.concatenate(
                            [v_main[pl.ds(ks, KW - TAIL), h, :],
                             v_tail[:, h, :]], axis=0)
                    q_blk = q_buf[pl.ds(qs, QB), pl.ds(DH * h, DH)]
                    s = lax.dot_general(q_blk, k_win,
                                        (((1,), (1,)), ((), ())),
                                        preferred_element_type=F32) * SCALE
                    qi = qs + lax.broadcasted_iota(jnp.int32, (QB, KW), 0)
                    ki = ks + lax.broadcasted_iota(jnp.int32, (QB, KW), 1)
                    s = jnp.where(jnp.abs(qi - ki) <= TAIL, s, -1e9)
                    m = jnp.max(s, axis=1, keepdims=True)
                    e = jnp.exp(s - m)
                    p = (e / jnp.sum(e, axis=1, keepdims=True)).astype(BF)
                    ctx_blk = lax.dot_general(p, v_win,
                                              (((1,), (0,)), ((), ())),
                                              preferred_element_type=F32)
                    ctx_buf[pl.ds(qs, QB), pl.ds(DH * h, DH)] = (
                        ctx_blk.astype(BF))

        for r in range(4):
            rows = pl.ds(512 * r, 512)
            pr = lax.dot_general(ctx_buf[rows, :], wo_bf[...],
                                 (((1,), (0,)), ((), ())),
                                 preferred_element_type=F32)
            p_r[rows, :] = pr[:, 0:512].astype(BF)
            p_l[rows, :] = pr[:, 512:].astype(BF)

        @pl.when(me == 0)
        def _():
            for slot, r in sorted(inflight):
                wait_slot(slot, r)

        right = lax.rem(me + 1, N_DEV)
        left = lax.rem(me + 3, N_DEV)

        def crows(c):
            return pl.ds(c * 512, 512)

        rings = ((p_r, recv_r, acc_r, ars_send_r, ars_recv_r, right, False),
                 (p_l, recv_l, acc_l, ars_send_l, ars_recv_l, left, True))
        ar_inflight = []

        def ar_step(rv, ssem, rsem, dev, step_idx, src, c_dst):
            snd = pltpu.make_async_remote_copy(
                src_ref=src, dst_ref=rv.at[crows(c_dst)],
                send_sem=ssem.at[step_idx], recv_sem=rsem.at[step_idx],
                device_id=(dev,), device_id_type=MESH)
            snd.start()
            ar_inflight.append(snd)

        def ar_wait_recv(rv, rsem, step_idx, c_in):
            pltpu.make_async_remote_copy(
                src_ref=rv.at[crows(c_in)], dst_ref=rv.at[crows(c_in)],
                send_sem=ars_send_r.at[step_idx], recv_sem=rsem.at[step_idx],
                device_id=(0,), device_id_type=MESH).wait_recv()

        for s in range(3):
            for (p, rv, ac, ssem, rsem, dev, is_l) in rings:
                c = lax.rem(me + s, 4) if is_l else lax.rem(me + 4 - s, 4)
                if s == 0:
                    ar_step(rv, ssem, rsem, dev, 0, p.at[crows(c)], c)
                else:
                    ar_wait_recv(rv, rsem, s - 1, c)
                    ac[s - 1, :, :] = p[crows(c), :] + rv[crows(c), :]
                    ar_step(rv, ssem, rsem, dev, s, ac.at[s - 1], c)
        for (p, rv, ac, ssem, rsem, dev, is_l) in rings:
            c_fin = lax.rem(me + 3, 4) if is_l else lax.rem(me + 1, 4)
            ar_wait_recv(rv, rsem, 2, c_fin)
            rv[crows(c_fin), :] = p[crows(c_fin), :] + rv[crows(c_fin), :]
        for s in range(3):
            for (p, rv, ac, ssem, rsem, dev, is_l) in rings:
                c = (lax.rem(me + 3 + s, 4) if is_l
                     else lax.rem(me + 5 - s, 4))
                if s > 0:
                    ar_wait_recv(rv, rsem, 3 + s - 1, c)
                ar_step(rv, ssem, rsem, dev, 3 + s, rv.at[crows(c)], c)
        for (p, rv, ac, ssem, rsem, dev, is_l) in rings:
            c_last = lax.rem(me + 6, 4) if is_l else lax.rem(me + 2, 4)
            ar_wait_recv(rv, rsem, 5, c_last)
        for snd in ar_inflight:
            snd.wait_send()

        for r in range(4):
            rows = pl.ds(512 * r, 512)
            out_ref[0, rows, 0:512] = recv_r[rows, :].astype(F32)
            out_ref[0, rows, 512:1024] = recv_l[rows, :].astype(F32)

    return pl.pallas_call(
        body,
        out_shape=jax.ShapeDtypeStruct((1, SQ, D), F32),
        in_specs=[pl.BlockSpec(memory_space=pl.ANY)] * 5,
        out_specs=pl.BlockSpec(memory_space=pltpu.MemorySpace.VMEM),
        scratch_shapes=[
            pltpu.VMEM((SQ, D), BF),
            pltpu.VMEM((D, D), BF),
            pltpu.VMEM((D, D), BF),
            pltpu.VMEM((SQ, H, DH), BF),
            pltpu.VMEM((SQ, H, DH), BF),
            pltpu.VMEM((TAIL, H, DH), BF),
            pltpu.VMEM((TAIL, H, DH), BF),
            pltpu.VMEM((512, D), F32),
            pltpu.VMEM((SUB, H, DH), F32),
            pltpu.VMEM((2, SQ, H, DH), BF),
            pltpu.VMEM((TAIL, 32, DH), F32),
            pltpu.VMEM((TAIL, H, DH), BF),
            pltpu.VMEM((SQ, D), BF),
            pltpu.VMEM((SQ, 512), BF),
            pltpu.VMEM((SQ, 512), BF),
            pltpu.VMEM((SQ, 512), BF),
            pltpu.VMEM((SQ, 512), BF),
            pltpu.VMEM((2, 512, 512), BF),
            pltpu.VMEM((2, 512, 512), BF),
            pltpu.SemaphoreType.DMA,
            pltpu.SemaphoreType.DMA((2, 4)),
            pltpu.SemaphoreType.DMA,
            pltpu.SemaphoreType.DMA((4,)),
            pltpu.SemaphoreType.DMA((4,)),
            pltpu.SemaphoreType.DMA((4,)),
            pltpu.SemaphoreType.DMA,
            pltpu.SemaphoreType.DMA,
            pltpu.SemaphoreType.DMA((4,)),
            pltpu.SemaphoreType.DMA((4,)),
            pltpu.SemaphoreType.DMA((6,)),
            pltpu.SemaphoreType.DMA((6,)),
            pltpu.SemaphoreType.DMA((6,)),
            pltpu.SemaphoreType.DMA((6,)),
        ],
        compiler_params=pltpu.CompilerParams(
            vmem_limit_bytes=63 * 1024 * 1024),
    )(x, Wq, K_ext, V_ext, Wo)
